# Initial kernel scaffold; baseline (speedup 1.0000x reference)
import functools

import jax
import jax.numpy as jnp
from jax import lax
from jax.experimental import pallas as pl
from jax.experimental.pallas import tpu as pltpu
from jax.experimental.pallas import tpu_sc as plsc

N = 10000
E = 320000
D = 128
NREL = 6
NQ = 4
DQ = D // NQ
NK = NREL * N
CH = 128
NCHUNK = E // CH
NT = 16
ROWS_PER_TILE = NK // NT


def _sc_body(xs_hbm, src_hbm, dst_hbm, et_hbm, zeros_hbm, ones_hbm,
             s_out, c_out,
             acc, src_v, dst_v, et_v, gidx_v, key_v, rows_v, sem):
  tid = lax.axis_index("s")
  cid = lax.axis_index("c")
  r0 = tid * ROWS_PER_TILE

  def zero_acc():
    pltpu.sync_copy(zeros_hbm, acc.at[pl.ds(r0, ROWS_PER_TILE)])

  def edge_loop(gather, q):
    n_i = (NCHUNK - tid + NT - 1) // NT

    @pl.loop(0, n_i)
    def _(i):
      base = (tid + i * NT) * CH
      pltpu.sync_copy(dst_hbm.at[pl.ds(base, CH)], dst_v)
      pltpu.sync_copy(et_hbm.at[pl.ds(base, CH)], et_v)
      if gather:
        pltpu.sync_copy(src_hbm.at[pl.ds(base, CH)], src_v)
      for j in range(CH // 16):
        sl = pl.ds(j * 16, 16)
        key_v[sl] = et_v[sl] * N + dst_v[sl]
        if gather:
          gidx_v[sl] = src_v[sl] + q * N
      if gather:
        pltpu.async_copy(xs_hbm.at[gidx_v], rows_v, sem).wait()
      pltpu.sync_copy(rows_v, acc.at[key_v], add=True)

  for p in range(2):
    q = 2 * cid + p
    zero_acc()
    plsc.subcore_barrier()
    edge_loop(True, q)
    plsc.subcore_barrier()
    pltpu.sync_copy(acc.at[pl.ds(r0, ROWS_PER_TILE)],
                    s_out.at[pl.ds(q * NK + r0, ROWS_PER_TILE)])

  @pl.when(cid == 1)
  def _():
    zero_acc()
    plsc.subcore_barrier()
    pltpu.sync_copy(ones_hbm, rows_v)
    edge_loop(False, 0)
    plsc.subcore_barrier()
    pltpu.sync_copy(acc.at[pl.ds(r0, ROWS_PER_TILE)],
                    c_out.at[pl.ds(r0, ROWS_PER_TILE)])


_sc_call = functools.partial(
    pl.kernel,
    out_type=(
        jax.ShapeDtypeStruct((NQ * NK, DQ), jnp.float32),
        jax.ShapeDtypeStruct((NK, DQ), jnp.float32),
    ),
    mesh=plsc.VectorSubcoreMesh(core_axis_name="c", subcore_axis_name="s"),
    scratch_types=[
        pltpu.VMEM_SHARED((NK, DQ), jnp.float32),
        pltpu.VMEM((CH,), jnp.int32),
        pltpu.VMEM((CH,), jnp.int32),
        pltpu.VMEM((CH,), jnp.int32),
        pltpu.VMEM((CH,), jnp.int32),
        pltpu.VMEM((CH,), jnp.int32),
        pltpu.VMEM((CH, DQ), jnp.float32),
        pltpu.SemaphoreType.DMA,
    ],
)(_sc_body)


BN = 400


def _tc_body(s_ref, c_ref, x_ref, w_ref, root_ref, bias_ref, o_ref):
  acc = jnp.dot(x_ref[...], root_ref[...],
                preferred_element_type=jnp.float32) + bias_ref[...]
  for r in range(NREL):
    cnt = jnp.maximum(c_ref[:, r], 1.0)[:, None]
    mean = jnp.concatenate([s_ref[q, r] for q in range(NQ)], axis=-1) / cnt
    acc = acc + jnp.dot(mean, w_ref[r], preferred_element_type=jnp.float32)
  o_ref[...] = acc


def _tc_call(s4, c2t, x, W, root, bias2):
  return pl.pallas_call(
      _tc_body,
      grid=(N // BN,),
      in_specs=[
          pl.BlockSpec((NQ, NREL, BN, DQ), lambda i: (0, 0, i, 0)),
          pl.BlockSpec((BN, NREL), lambda i: (i, 0)),
          pl.BlockSpec((BN, D), lambda i: (i, 0)),
          pl.BlockSpec((NREL, D, D), lambda i: (0, 0, 0)),
          pl.BlockSpec((D, D), lambda i: (0, 0)),
          pl.BlockSpec((1, D), lambda i: (0, 0)),
      ],
      out_specs=pl.BlockSpec((BN, D), lambda i: (i, 0)),
      out_shape=jax.ShapeDtypeStruct((N, D), jnp.float32),
  )(s4, c2t, x, W, root, bias2)


@jax.jit
def kernel(x, edge_index, edge_type, W, root, bias):
  src = edge_index[0].astype(jnp.int32)
  dst = edge_index[1].astype(jnp.int32)
  et = edge_type.astype(jnp.int32)
  xs = x.reshape(N, NQ, DQ).transpose(1, 0, 2).reshape(NQ * N, DQ)
  zeros = jnp.zeros((ROWS_PER_TILE, DQ), jnp.float32)
  ones = jnp.ones((CH, DQ), jnp.float32)

  s_flat, c32 = _sc_call(xs, src, dst, et, zeros, ones)

  s4 = s_flat.reshape(NQ, NREL, N, DQ)
  c2t = c32[:, 0].reshape(NREL, N).T
  return _tc_call(s4, c2t, x, W, root, bias.reshape(1, D))


# speedup vs baseline: 5.6978x; 5.6978x over previous
import functools

import jax
import jax.numpy as jnp
from jax import lax
from jax.experimental import pallas as pl
from jax.experimental.pallas import tpu as pltpu
from jax.experimental.pallas import tpu_sc as plsc

N = 10000
E = 320000
D = 128
NREL = 6
NQ = 4
DQ = D // NQ
NK = NREL * N
NKP = 60032
CH = 128
NCHUNK = E // CH
NT = 16
ROWS_PER_TILE = NKP // NT


def _sc_body(xs_hbm, src_hbm, dst_hbm, et_hbm, zeros_hbm, ones_hbm,
             s_out, c_out,
             acc, src_v, dst_v, et_v, gidx_v, key_v, rows_v, sem):
  tid = lax.axis_index("s")
  cid = lax.axis_index("c")
  r0 = tid * ROWS_PER_TILE

  def zero_acc():
    pltpu.sync_copy(zeros_hbm, acc.at[pl.ds(r0, ROWS_PER_TILE)])

  def edge_loop(gather, q):
    n_i = (NCHUNK - tid + NT - 1) // NT

    @pl.loop(0, n_i)
    def _(i):
      base = (tid + i * NT) * CH
      pltpu.sync_copy(dst_hbm.at[pl.ds(base, CH)], dst_v)
      pltpu.sync_copy(et_hbm.at[pl.ds(base, CH)], et_v)
      if gather:
        pltpu.sync_copy(src_hbm.at[pl.ds(base, CH)], src_v)
      for j in range(CH // 16):
        sl = pl.ds(j * 16, 16)
        key_v[sl] = et_v[sl] * N + dst_v[sl]
        if gather:
          gidx_v[sl] = src_v[sl] + q * N
      if gather:
        pltpu.async_copy(xs_hbm.at[gidx_v], rows_v, sem).wait()
      pltpu.sync_copy(rows_v, acc.at[key_v], add=True)

  for p in range(2):
    q = 2 * cid + p
    zero_acc()
    plsc.subcore_barrier()
    edge_loop(True, q)
    plsc.subcore_barrier()
    pltpu.sync_copy(acc.at[pl.ds(r0, ROWS_PER_TILE)],
                    s_out.at[pl.ds(q * NKP + r0, ROWS_PER_TILE)])

  @pl.when(cid == 1)
  def _():
    zero_acc()
    plsc.subcore_barrier()
    pltpu.sync_copy(ones_hbm, rows_v)
    edge_loop(False, 0)
    plsc.subcore_barrier()
    pltpu.sync_copy(acc.at[pl.ds(r0, ROWS_PER_TILE)],
                    c_out.at[pl.ds(r0, ROWS_PER_TILE)])


_sc_call = functools.partial(
    pl.kernel,
    out_type=(
        jax.ShapeDtypeStruct((NQ * NKP, DQ), jnp.float32),
        jax.ShapeDtypeStruct((NKP, DQ), jnp.float32),
    ),
    mesh=plsc.VectorSubcoreMesh(core_axis_name="c", subcore_axis_name="s"),
    compiler_params=pltpu.CompilerParams(use_tc_tiling_on_sc=False),
    scratch_types=[
        pltpu.VMEM_SHARED((NKP, DQ), jnp.float32),
        pltpu.VMEM((CH,), jnp.int32),
        pltpu.VMEM((CH,), jnp.int32),
        pltpu.VMEM((CH,), jnp.int32),
        pltpu.VMEM((CH,), jnp.int32),
        pltpu.VMEM((CH,), jnp.int32),
        pltpu.VMEM((CH, DQ), jnp.float32),
        pltpu.SemaphoreType.DMA,
    ],
)(_sc_body)


BN = 400


def _tc_body(s_ref, c_ref, x_ref, w_ref, root_ref, bias_ref, o_ref):
  acc = jnp.dot(x_ref[...], root_ref[...],
                preferred_element_type=jnp.float32) + bias_ref[...]
  for r in range(NREL):
    cnt = jnp.maximum(c_ref[:, r], 1.0)[:, None]
    mean = jnp.concatenate([s_ref[q, r] for q in range(NQ)], axis=-1) / cnt
    acc = acc + jnp.dot(mean, w_ref[r], preferred_element_type=jnp.float32)
  o_ref[...] = acc


def _tc_call(s4, c2t, x, W, root, bias2):
  return pl.pallas_call(
      _tc_body,
      grid=(N // BN,),
      in_specs=[
          pl.BlockSpec((NQ, NREL, BN, DQ), lambda i: (0, 0, i, 0)),
          pl.BlockSpec((BN, NREL), lambda i: (i, 0)),
          pl.BlockSpec((BN, D), lambda i: (i, 0)),
          pl.BlockSpec((NREL, D, D), lambda i: (0, 0, 0)),
          pl.BlockSpec((D, D), lambda i: (0, 0)),
          pl.BlockSpec((1, D), lambda i: (0, 0)),
      ],
      out_specs=pl.BlockSpec((BN, D), lambda i: (i, 0)),
      out_shape=jax.ShapeDtypeStruct((N, D), jnp.float32),
  )(s4, c2t, x, W, root, bias2)


@jax.jit
def kernel(x, edge_index, edge_type, W, root, bias):
  src = edge_index[0].astype(jnp.int32)
  dst = edge_index[1].astype(jnp.int32)
  et = edge_type.astype(jnp.int32)
  xs = x.reshape(N, NQ, DQ).transpose(1, 0, 2).reshape(NQ * N, DQ)
  zeros = jnp.zeros((ROWS_PER_TILE, DQ), jnp.float32)
  ones = jnp.ones((CH, DQ), jnp.float32)

  s_flat, c32 = _sc_call(xs, src, dst, et, zeros, ones)

  s4 = s_flat.reshape(NQ, NKP, DQ)[:, :NK].reshape(NQ, NREL, N, DQ)
  c2t = c32[:NK, 0].reshape(NREL, N).T
  return _tc_call(s4, c2t, x, W, root, bias.reshape(1, D))


# speedup vs baseline: 11.1435x; 1.9558x over previous
import functools

import jax
import jax.numpy as jnp
from jax import lax
from jax.experimental import pallas as pl
from jax.experimental.pallas import tpu as pltpu
from jax.experimental.pallas import tpu_sc as plsc

N = 10000
E = 320000
D = 128
NREL = 6
NQ = 4
DQ = D // NQ
NK = NREL * N
NKP = 60032
CH = 128
NSUB = 4
SCH = CH * NSUB
NSUP = E // SCH
NB = 2
NT = 16
ROWS_PER_TILE = NKP // NT
CNT0 = NSUP // 2


def _sc_body(xs_hbm, ed_hbm, zeros_hbm, ones_hbm, s_out, c_out,
             acc, ed_v, key2, rows_v, gsem, ssem):
  tid = lax.axis_index("s")
  cid = lax.axis_index("c")
  r0 = tid * ROWS_PER_TILE

  def compute_keys(sub):
    for j in range(CH // 16):
      sl = pl.ds(j * 16, 16)
      e0 = sub * CH + j * 16
      key2[sub, sl] = (ed_v[pl.ds(2 * SCH + e0, 16)] * N
                       + ed_v[pl.ds(SCH + e0, 16)])

  def drain(sd):
    for d in sd:
      if d is not None:
        d.wait()

  def quarter_pass(q):
    n_i = (NSUP - tid + NT - 1) // NT

    @pl.loop(0, n_i)
    def _(i):
      chunk = tid + i * NT
      pltpu.sync_copy(ed_hbm.at[chunk], ed_v)
      for sub in range(NSUB):
        compute_keys(sub)
        for j in range(CH // 16):
          e0 = sub * CH + j * 16
          sl = pl.ds(e0, 16)
          ed_v[sl] = ed_v[sl] + q * N
      gd = [None] * NSUB
      sd = [None] * NSUB
      for sub in range(NSUB):
        b = sub % NB
        if sub >= NB:
          sd[sub - NB].wait()
        gd[sub] = pltpu.async_copy(
            xs_hbm.at[ed_v.at[pl.ds(sub * CH, CH)]], rows_v.at[b],
            gsem.at[b])
        if sub >= 1:
          pb = (sub - 1) % NB
          gd[sub - 1].wait()
          sd[sub - 1] = pltpu.async_copy(
              rows_v.at[pb], acc.at[key2.at[sub - 1]], ssem.at[pb], add=True)
      gd[NSUB - 1].wait()
      sd[NSUB - 1] = pltpu.async_copy(
          rows_v.at[(NSUB - 1) % NB], acc.at[key2.at[NSUB - 1]],
          ssem.at[(NSUB - 1) % NB], add=True)
      drain(sd[max(0, NSUB - NB):])

  def count_pass():
    start = cid * CNT0
    n_c = CNT0 + cid
    n_i = (n_c - tid + NT - 1) // NT
    pltpu.sync_copy(ones_hbm, rows_v.at[0])

    @pl.loop(0, n_i)
    def _(i):
      chunk = start + tid + i * NT
      pltpu.sync_copy(ed_hbm.at[chunk], ed_v)
      sd = [None] * NSUB
      for sub in range(NSUB):
        compute_keys(sub)
        sd[sub] = pltpu.async_copy(
            rows_v.at[0], acc.at[key2.at[sub]], ssem.at[sub % NB], add=True)
      drain(sd)

  def zero_acc():
    pltpu.sync_copy(zeros_hbm, acc.at[pl.ds(r0, ROWS_PER_TILE)])

  for p in range(2):
    q = 2 * cid + p
    zero_acc()
    plsc.subcore_barrier()
    quarter_pass(q)
    plsc.subcore_barrier()
    pltpu.sync_copy(acc.at[pl.ds(r0, ROWS_PER_TILE)],
                    s_out.at[pl.ds(q * NKP + r0, ROWS_PER_TILE)])

  zero_acc()
  plsc.subcore_barrier()
  count_pass()
  plsc.subcore_barrier()
  pltpu.sync_copy(acc.at[pl.ds(r0, ROWS_PER_TILE)],
                  c_out.at[pl.ds(cid * NKP + r0, ROWS_PER_TILE)])


_sc_call = functools.partial(
    pl.kernel,
    out_type=(
        jax.ShapeDtypeStruct((NQ * NKP, DQ), jnp.float32),
        jax.ShapeDtypeStruct((2 * NKP, DQ), jnp.float32),
    ),
    mesh=plsc.VectorSubcoreMesh(core_axis_name="c", subcore_axis_name="s"),
    compiler_params=pltpu.CompilerParams(use_tc_tiling_on_sc=False),
    scratch_types=[
        pltpu.VMEM_SHARED((NKP, DQ), jnp.float32),
        pltpu.VMEM((3 * SCH,), jnp.int32),
        pltpu.VMEM((NSUB, CH), jnp.int32),
        pltpu.VMEM((NB, CH, DQ), jnp.float32),
        pltpu.SemaphoreType.DMA((NB,)),
        pltpu.SemaphoreType.DMA((NB,)),
    ],
)(_sc_body)


BN = 400


def _tc_body(s_ref, c_ref, x_ref, w_ref, root_ref, bias_ref, o_ref):
  acc = jnp.dot(x_ref[...], root_ref[...],
                preferred_element_type=jnp.float32) + bias_ref[...]
  for r in range(NREL):
    cnt = jnp.maximum(c_ref[:, r] + c_ref[:, NREL + r], 1.0)[:, None]
    mean = jnp.concatenate([s_ref[q, r] for q in range(NQ)], axis=-1) / cnt
    acc = acc + jnp.dot(mean, w_ref[r], preferred_element_type=jnp.float32)
  o_ref[...] = acc


def _tc_call(s4, cc, x, W, root, bias2):
  return pl.pallas_call(
      _tc_body,
      grid=(N // BN,),
      in_specs=[
          pl.BlockSpec((NQ, NREL, BN, DQ), lambda i: (0, 0, i, 0)),
          pl.BlockSpec((BN, 2 * NREL), lambda i: (i, 0)),
          pl.BlockSpec((BN, D), lambda i: (i, 0)),
          pl.BlockSpec((NREL, D, D), lambda i: (0, 0, 0)),
          pl.BlockSpec((D, D), lambda i: (0, 0)),
          pl.BlockSpec((1, D), lambda i: (0, 0)),
      ],
      out_specs=pl.BlockSpec((BN, D), lambda i: (i, 0)),
      out_shape=jax.ShapeDtypeStruct((N, D), jnp.float32),
  )(s4, cc, x, W, root, bias2)


@jax.jit
def kernel(x, edge_index, edge_type, W, root, bias):
  src = edge_index[0].astype(jnp.int32)
  dst = edge_index[1].astype(jnp.int32)
  et = edge_type.astype(jnp.int32)
  ed = jnp.concatenate(
      [src.reshape(NSUP, SCH), dst.reshape(NSUP, SCH), et.reshape(NSUP, SCH)],
      axis=1)
  xs = x.reshape(N, NQ, DQ).transpose(1, 0, 2).reshape(NQ * N, DQ)
  zeros = jnp.zeros((ROWS_PER_TILE, DQ), jnp.float32)
  ones = jnp.ones((CH, DQ), jnp.float32)

  s_flat, c2 = _sc_call(xs, ed, zeros, ones)

  s4 = s_flat.reshape(NQ, NKP, DQ)[:, :NK].reshape(NQ, NREL, N, DQ)
  cc = c2[:, 0].reshape(2, NKP)[:, :NK].reshape(2 * NREL, N).T
  return _tc_call(s4, cc, x, W, root, bias.reshape(1, D))


# speedup vs baseline: 12.5850x; 1.1294x over previous
import functools

import jax
import jax.numpy as jnp
from jax import lax
from jax.experimental import pallas as pl
from jax.experimental.pallas import tpu as pltpu
from jax.experimental.pallas import tpu_sc as plsc

N = 10000
E = 320000
D = 128
NREL = 6
NQ = 4
DQ = D // NQ
NK = NREL * N
NKP = 60032
CH = 128
NSUB = 4
SCH = CH * NSUB
NSUP = E // SCH
NB = 2
NT = 16
ROWS_PER_TILE = NKP // NT
LAST_ROWS = NK - 15 * ROWS_PER_TILE
CW = 8
CNT0 = NSUP // 2


def _sc_body(xs_hbm, ei_hbm, et_hbm, zeros_hbm, ones_hbm, s_out, c_out,
             acc, src_v, dst_v, et_v, key2, rows_v, gsem, ssem, isem):
  tid = lax.axis_index("s")
  cid = lax.axis_index("c")
  r0 = tid * ROWS_PER_TILE

  def load_idx(base):
    d1 = pltpu.async_copy(ei_hbm.at[0, pl.ds(base, SCH)], src_v, isem)
    d2 = pltpu.async_copy(ei_hbm.at[1, pl.ds(base, SCH)], dst_v, isem)
    d3 = pltpu.async_copy(et_hbm.at[pl.ds(base, SCH)], et_v, isem)
    d1.wait()
    d2.wait()
    d3.wait()

  def compute_keys(sub):
    for j in range(CH // 16):
      sl = pl.ds(j * 16, 16)
      e0 = pl.ds(sub * CH + j * 16, 16)
      key2[sub, sl] = et_v[e0] * N + dst_v[e0]

  def drain(sd):
    for d in sd:
      if d is not None:
        d.wait()

  def quarter_pass(q):
    n_i = (NSUP - tid + NT - 1) // NT

    @pl.loop(0, n_i)
    def _(i):
      base = (tid + i * NT) * SCH
      load_idx(base)
      for sub in range(NSUB):
        compute_keys(sub)
        for j in range(CH // 16):
          sl = pl.ds(sub * CH + j * 16, 16)
          src_v[sl] = src_v[sl] + q * N
      gd = [None] * NSUB
      sd = [None] * NSUB
      for sub in range(NSUB):
        b = sub % NB
        if sub >= NB:
          sd[sub - NB].wait()
        gd[sub] = pltpu.async_copy(
            xs_hbm.at[src_v.at[pl.ds(sub * CH, CH)]], rows_v.at[b],
            gsem.at[b])
        if sub >= 1:
          pb = (sub - 1) % NB
          gd[sub - 1].wait()
          sd[sub - 1] = pltpu.async_copy(
              rows_v.at[pb], acc.at[key2.at[sub - 1]], ssem.at[pb], add=True)
      gd[NSUB - 1].wait()
      sd[NSUB - 1] = pltpu.async_copy(
          rows_v.at[(NSUB - 1) % NB], acc.at[key2.at[NSUB - 1]],
          ssem.at[(NSUB - 1) % NB], add=True)
      drain(sd[NSUB - NB:])

  def count_pass():
    start = cid * CNT0
    n_c = CNT0 + cid
    n_i = (n_c - tid + NT - 1) // NT
    pltpu.sync_copy(ones_hbm, rows_v.at[0])

    @pl.loop(0, n_i)
    def _(i):
      base = (start + tid + i * NT) * SCH
      load_idx(base)
      sd = [None] * NSUB
      for sub in range(NSUB):
        compute_keys(sub)
        sd[sub] = pltpu.async_copy(
            rows_v.at[0], acc.at[key2.at[sub]], ssem.at[sub % NB], add=True)
      drain(sd)

  def zero_acc():
    pltpu.sync_copy(zeros_hbm, acc.at[pl.ds(r0, ROWS_PER_TILE)])

  def writeback(dst, row_base, width):
    @pl.when(tid < NT - 1)
    def _():
      pltpu.sync_copy(acc.at[pl.ds(r0, ROWS_PER_TILE), pl.ds(0, width)],
                      dst.at[pl.ds(row_base + r0, ROWS_PER_TILE)])

    @pl.when(tid == NT - 1)
    def _():
      pltpu.sync_copy(acc.at[pl.ds(r0, LAST_ROWS), pl.ds(0, width)],
                      dst.at[pl.ds(row_base + r0, LAST_ROWS)])

  for p in range(2):
    q = 2 * cid + p
    zero_acc()
    plsc.subcore_barrier()
    quarter_pass(q)
    plsc.subcore_barrier()
    writeback(s_out, q * NK, DQ)

  zero_acc()
  plsc.subcore_barrier()
  count_pass()
  plsc.subcore_barrier()
  writeback(c_out, cid * NK, CW)


_sc_call = functools.partial(
    pl.kernel,
    out_type=(
        jax.ShapeDtypeStruct((NQ * NK, DQ), jnp.float32),
        jax.ShapeDtypeStruct((2 * NK, CW), jnp.float32),
    ),
    mesh=plsc.VectorSubcoreMesh(core_axis_name="c", subcore_axis_name="s"),
    compiler_params=pltpu.CompilerParams(use_tc_tiling_on_sc=False),
    scratch_types=[
        pltpu.VMEM_SHARED((NKP, DQ), jnp.float32),
        pltpu.VMEM((SCH,), jnp.int32),
        pltpu.VMEM((SCH,), jnp.int32),
        pltpu.VMEM((SCH,), jnp.int32),
        pltpu.VMEM((NSUB, CH), jnp.int32),
        pltpu.VMEM((NB, CH, DQ), jnp.float32),
        pltpu.SemaphoreType.DMA((NB,)),
        pltpu.SemaphoreType.DMA((NB,)),
        pltpu.SemaphoreType.DMA,
    ],
)(_sc_body)


BN = 400


def _tc_body(s_ref, c_ref, x_ref, w_ref, root_ref, bias_ref, o_ref):
  acc = jnp.dot(x_ref[...], root_ref[...],
                preferred_element_type=jnp.float32) + bias_ref[...]
  for r in range(NREL):
    cnt = jnp.maximum(c_ref[0, r, :, 0] + c_ref[1, r, :, 0], 1.0)[:, None]
    mean = jnp.concatenate([s_ref[q, r] for q in range(NQ)], axis=-1) / cnt
    acc = acc + jnp.dot(mean, w_ref[r], preferred_element_type=jnp.float32)
  o_ref[...] = acc


def _tc_call(s4, c4, x, W, root, bias2):
  return pl.pallas_call(
      _tc_body,
      grid=(N // BN,),
      in_specs=[
          pl.BlockSpec((NQ, NREL, BN, DQ), lambda i: (0, 0, i, 0)),
          pl.BlockSpec((2, NREL, BN, CW), lambda i: (0, 0, i, 0)),
          pl.BlockSpec((BN, D), lambda i: (i, 0)),
          pl.BlockSpec((NREL, D, D), lambda i: (0, 0, 0)),
          pl.BlockSpec((D, D), lambda i: (0, 0)),
          pl.BlockSpec((1, D), lambda i: (0, 0)),
      ],
      out_specs=pl.BlockSpec((BN, D), lambda i: (i, 0)),
      out_shape=jax.ShapeDtypeStruct((N, D), jnp.float32),
  )(s4, c4, x, W, root, bias2)


@jax.jit
def kernel(x, edge_index, edge_type, W, root, bias):
  ei = edge_index.astype(jnp.int32)
  et = edge_type.astype(jnp.int32)
  xs = x.reshape(N, NQ, DQ).transpose(1, 0, 2).reshape(NQ * N, DQ)
  zeros = jnp.zeros((ROWS_PER_TILE, DQ), jnp.float32)
  ones = jnp.ones((CH, DQ), jnp.float32)

  s_flat, c_flat = _sc_call(xs, ei, et, zeros, ones)

  s4 = s_flat.reshape(NQ, NREL, N, DQ)
  c4 = c_flat.reshape(2, NREL, N, CW)
  return _tc_call(s4, c4, x, W, root, bias.reshape(1, D))


# speedup vs baseline: 16.0156x; 1.2726x over previous
import functools

import jax
import jax.numpy as jnp
from jax import lax
from jax.experimental import pallas as pl
from jax.experimental.pallas import tpu as pltpu
from jax.experimental.pallas import tpu_sc as plsc

N = 10000
E = 320000
D = 128
NREL = 6
NQ = 4
DQ = D // NQ
NK = NREL * N
NKP = 60032
CH = 128
NSUB = 2
SCH = CH * NSUB
NSUP = E // SCH
NT = 16
ROWS_PER_TILE = NKP // NT
LAST_ROWS = NK - 15 * ROWS_PER_TILE
CW = 8
CNT0 = NSUP // 2


def _sc_body(xs_hbm, ed_hbm, zeros_hbm, ones_hbm, s_out, c_out,
             acc, ed_v, key2, rows_v, gsem, ssem, isem):
  tid = lax.axis_index("s")
  cid = lax.axis_index("c")
  r0 = tid * ROWS_PER_TILE

  def fire_idx(chunk, par):
    return pltpu.async_copy(ed_hbm.at[chunk], ed_v.at[par], isem.at[par])

  def wait_idx(par):
    pltpu.make_async_copy(ed_hbm.at[0], ed_v.at[par], isem.at[par]).wait()

  def wait_scatter(b):
    pltpu.make_async_copy(rows_v.at[b], acc.at[pl.ds(0, CH)],
                          ssem.at[b]).wait()

  def compute_keys(par, q, with_gidx):
    for sub in range(NSUB):
      for j in range(CH // 16):
        sl = pl.ds(j * 16, 16)
        e0 = sub * CH + j * 16
        key2[par, sub, sl] = (ed_v[par, pl.ds(2 * SCH + e0, 16)] * N
                              + ed_v[par, pl.ds(SCH + e0, 16)])
        if with_gidx:
          es = pl.ds(e0, 16)
          ed_v[par, es] = ed_v[par, es] + q * N

  def quarter_pass(q):
    n_i = (NSUP - tid + NT - 1) // NT

    fire_idx(tid, 0)

    def body(i, par, first):
      @pl.when(i + 1 < n_i)
      def _():
        fire_idx(tid + (i + 1) * NT, 1 - par)
      wait_idx(par)
      compute_keys(par, q, True)
      gd = [None] * NSUB
      for b in range(NSUB):
        if not first:
          wait_scatter(b)
        gd[b] = pltpu.async_copy(
            xs_hbm.at[ed_v.at[par, pl.ds(b * CH, CH)]], rows_v.at[b],
            gsem.at[b])
      for b in range(NSUB):
        gd[b].wait()
        pltpu.async_copy(rows_v.at[b], acc.at[key2.at[par, b]], ssem.at[b],
                         add=True)

    body(0, 0, True)

    @pl.loop(1, n_i)
    def _(i):
      body(i, i % 2, False)

    for b in range(NSUB):
      wait_scatter(b)

  def count_pass():
    start = cid * CNT0
    n_c = CNT0 + cid * (NSUP - 2 * CNT0)
    n_i = (n_c - tid + NT - 1) // NT
    pltpu.sync_copy(ones_hbm, rows_v.at[0])

    fire_idx(start + tid, 0)

    def body(i, par, first):
      @pl.when(i + 1 < n_i)
      def _():
        fire_idx(start + tid + (i + 1) * NT, 1 - par)
      wait_idx(par)
      compute_keys(par, 0, False)
      for b in range(NSUB):
        if not first:
          wait_scatter(b)
        pltpu.async_copy(rows_v.at[0], acc.at[key2.at[par, b]], ssem.at[b],
                         add=True)

    body(0, 0, True)

    @pl.loop(1, n_i)
    def _(i):
      body(i, i % 2, False)

    for b in range(NSUB):
      wait_scatter(b)

  def zero_acc():
    pltpu.sync_copy(zeros_hbm, acc.at[pl.ds(r0, ROWS_PER_TILE)])

  def writeback(dst, row_base, width):
    @pl.when(tid < NT - 1)
    def _():
      pltpu.sync_copy(acc.at[pl.ds(r0, ROWS_PER_TILE), pl.ds(0, width)],
                      dst.at[pl.ds(row_base + r0, ROWS_PER_TILE)])

    @pl.when(tid == NT - 1)
    def _():
      pltpu.sync_copy(acc.at[pl.ds(r0, LAST_ROWS), pl.ds(0, width)],
                      dst.at[pl.ds(row_base + r0, LAST_ROWS)])

  def writeback_stripe(dst, col):
    @pl.when(tid < NT - 1)
    def _():
      pltpu.sync_copy(acc.at[pl.ds(r0, ROWS_PER_TILE)],
                      dst.at[pl.ds(r0, ROWS_PER_TILE), pl.ds(col, DQ)])

    @pl.when(tid == NT - 1)
    def _():
      pltpu.sync_copy(acc.at[pl.ds(r0, LAST_ROWS)],
                      dst.at[pl.ds(r0, LAST_ROWS), pl.ds(col, DQ)])

  for p in range(2):
    q = 2 * cid + p
    zero_acc()
    plsc.subcore_barrier()
    quarter_pass(q)
    plsc.subcore_barrier()
    for c in range(2):
      @pl.when(cid == c)
      def _(col=DQ * (2 * c + p)):
        writeback_stripe(s_out, col)

  zero_acc()
  plsc.subcore_barrier()
  count_pass()
  plsc.subcore_barrier()
  writeback(c_out, cid * NK, CW)


_sc_call = functools.partial(
    pl.kernel,
    out_type=(
        jax.ShapeDtypeStruct((NK, D), jnp.float32),
        jax.ShapeDtypeStruct((2 * NK, CW), jnp.float32),
    ),
    mesh=plsc.VectorSubcoreMesh(core_axis_name="c", subcore_axis_name="s"),
    compiler_params=pltpu.CompilerParams(use_tc_tiling_on_sc=False),
    scratch_types=[
        pltpu.VMEM_SHARED((NKP, DQ), jnp.float32),
        pltpu.VMEM((2, 3 * SCH), jnp.int32),
        pltpu.VMEM((2, NSUB, CH), jnp.int32),
        pltpu.VMEM((NSUB, CH, DQ), jnp.float32),
        pltpu.SemaphoreType.DMA((NSUB,)),
        pltpu.SemaphoreType.DMA((NSUB,)),
        pltpu.SemaphoreType.DMA((2,)),
    ],
)(_sc_body)


BN = 400


def _tc_body(s_ref, c_ref, x_ref, w_ref, root_ref, bias_ref, o_ref):
  acc = jnp.dot(x_ref[...], root_ref[...],
                preferred_element_type=jnp.float32) + bias_ref[...]
  for r in range(NREL):
    cnt = jnp.maximum(c_ref[:, r] + c_ref[:, NREL + r], 1.0)[:, None]
    acc = acc + jnp.dot(s_ref[r] / cnt, w_ref[r],
                        preferred_element_type=jnp.float32)
  o_ref[...] = acc


def _tc_call(s3, c12, x, W, root, bias2):
  return pl.pallas_call(
      _tc_body,
      grid=(N // BN,),
      in_specs=[
          pl.BlockSpec((NREL, BN, D), lambda i: (0, i, 0)),
          pl.BlockSpec((BN, 2 * NREL), lambda i: (i, 0)),
          pl.BlockSpec((BN, D), lambda i: (i, 0)),
          pl.BlockSpec((NREL, D, D), lambda i: (0, 0, 0)),
          pl.BlockSpec((D, D), lambda i: (0, 0)),
          pl.BlockSpec((1, D), lambda i: (0, 0)),
      ],
      out_specs=pl.BlockSpec((BN, D), lambda i: (i, 0)),
      out_shape=jax.ShapeDtypeStruct((N, D), jnp.float32),
  )(s3, c12, x, W, root, bias2)


@jax.jit
def kernel(x, edge_index, edge_type, W, root, bias):
  src = edge_index[0].astype(jnp.int32)
  dst = edge_index[1].astype(jnp.int32)
  et = edge_type.astype(jnp.int32)
  ed = jnp.concatenate(
      [src.reshape(NSUP, SCH), dst.reshape(NSUP, SCH), et.reshape(NSUP, SCH)],
      axis=1)
  xs = x.reshape(N, NQ, DQ).transpose(1, 0, 2).reshape(NQ * N, DQ)
  zeros = jnp.zeros((ROWS_PER_TILE, DQ), jnp.float32)
  ones = jnp.ones((CH, DQ), jnp.float32)

  s_grid, c_flat = _sc_call(xs, ed, zeros, ones)

  s3 = s_grid.reshape(NREL, N, D)
  c12 = c_flat[:, 0].reshape(2 * NREL, N).T
  return _tc_call(s3, c12, x, W, root, bias.reshape(1, D))
